# baseline (speedup 1.0000x reference)
import functools

import jax
import jax.numpy as jnp
from jax import lax
from jax.experimental import pallas as pl
from jax.experimental.pallas import tpu as pltpu
from jax.experimental.pallas import tpu_sc as plsc

NC = 2
NS = 16
LANES = 16
GC = 128


def _body(n_factors, b_per_w, n_chunks,
          user_hbm, item_hbm, uf_hbm, vf_hbm, out_hbm,
          uidx, iidx, urows, vrows, out_v, sem0, sem1):
    c = lax.axis_index("c")
    s = lax.axis_index("s")
    wid = s * NC + c
    base = wid * b_per_w
    sems = [sem0, sem1]

    pltpu.sync_copy(user_hbm.at[pl.ds(base, b_per_w)], uidx)
    pltpu.sync_copy(item_hbm.at[pl.ds(base, b_per_w)], iidx)

    def fire(k):
        slot = k % 2
        hu = pltpu.async_copy(uf_hbm.at[uidx.at[pl.ds(k * GC, GC)]],
                              urows.at[slot], sems[slot])
        hv = pltpu.async_copy(vf_hbm.at[iidx.at[pl.ds(k * GC, GC)]],
                              vrows.at[slot], sems[slot])
        return (hu, hv)

    lane = lax.iota(jnp.int32, LANES)
    n_vec = n_factors // LANES

    handles = [fire(0), fire(1)]
    for k in range(n_chunks):
        slot = k % 2
        hu, hv = handles[slot]
        hu.wait()
        hv.wait()

        def group(g, carry, slot=slot, k=k):
            ov = jnp.zeros((LANES,), jnp.float32)
            for l in range(LANES):
                r = g * LANES + l
                acc = (urows[slot, r, pl.ds(0, LANES)]
                       * vrows[slot, r, pl.ds(0, LANES)])
                for j in range(1, n_vec):
                    acc = acc + (urows[slot, r, pl.ds(j * LANES, LANES)]
                                 * vrows[slot, r, pl.ds(j * LANES, LANES)])
                ov = jnp.where(lane == l, jnp.sum(acc), ov)
            out_v[pl.ds(k * GC + g * LANES, LANES)] = ov
            return carry

        lax.fori_loop(0, GC // LANES, group, 0)
        if k + 2 < n_chunks:
            handles[slot] = fire(k + 2)

    pltpu.sync_copy(out_v, out_hbm.at[pl.ds(base, b_per_w)])


def kernel(user, item, user_factors, item_factors):
    batch = user.shape[0]
    n_rows, n_factors = user_factors.shape
    nw = NC * NS
    b_per_w = batch // nw
    n_chunks = b_per_w // GC

    mesh = plsc.VectorSubcoreMesh(core_axis_name="c", subcore_axis_name="s")

    out = pl.kernel(
        functools.partial(_body, n_factors, b_per_w, n_chunks),
        out_type=jax.ShapeDtypeStruct((batch,), jnp.float32),
        mesh=mesh,
        scratch_types=[
            pltpu.VMEM((b_per_w,), jnp.int32),
            pltpu.VMEM((b_per_w,), jnp.int32),
            pltpu.VMEM((2, GC, n_factors), jnp.float32),
            pltpu.VMEM((2, GC, n_factors), jnp.float32),
            pltpu.VMEM((b_per_w,), jnp.float32),
            pltpu.SemaphoreType.DMA,
            pltpu.SemaphoreType.DMA,
        ],
        compiler_params=pltpu.CompilerParams(
            needs_layout_passes=False, use_tc_tiling_on_sc=False),
    )(user.astype(jnp.int32), item.astype(jnp.int32),
      user_factors, item_factors)
    return out


# speedup vs baseline: 1.0481x; 1.0481x over previous
import functools

import jax
import jax.numpy as jnp
from jax import lax
from jax.experimental import pallas as pl
from jax.experimental.pallas import tpu as pltpu
from jax.experimental.pallas import tpu_sc as plsc

NC = 2
NS = 16
LANES = 16
GC = 128
PADF = 128


def _body(n_factors, b_per_w, n_chunks,
          user_hbm, item_hbm, uf_hbm, vf_hbm, out_hbm,
          uidx, iidx, urows, vrows, out_v, sem0, sem1):
    c = lax.axis_index("c")
    s = lax.axis_index("s")
    wid = s * NC + c
    base = wid * b_per_w
    sems = [sem0, sem1]

    pltpu.sync_copy(user_hbm.at[pl.ds(base, b_per_w)], uidx)
    pltpu.sync_copy(item_hbm.at[pl.ds(base, b_per_w)], iidx)

    def fire(k):
        slot = k % 2
        hu = pltpu.async_copy(uf_hbm.at[uidx.at[pl.ds(k * GC, GC)]],
                              urows.at[slot], sems[slot])
        hv = pltpu.async_copy(vf_hbm.at[iidx.at[pl.ds(k * GC, GC)]],
                              vrows.at[slot], sems[slot])
        return (hu, hv)

    lane = lax.iota(jnp.int32, LANES)
    n_vec = n_factors // LANES

    handles = [fire(0), fire(1)]
    for k in range(n_chunks):
        slot = k % 2
        hu, hv = handles[slot]
        hu.wait()
        hv.wait()

        def group(g, carry, slot=slot, k=k):
            ov = jnp.zeros((LANES,), jnp.float32)
            for l in range(LANES):
                r = g * LANES + l
                acc = (urows[slot, r, pl.ds(0, LANES)]
                       * vrows[slot, r, pl.ds(0, LANES)])
                for j in range(1, n_vec):
                    acc = acc + (urows[slot, r, pl.ds(j * LANES, LANES)]
                                 * vrows[slot, r, pl.ds(j * LANES, LANES)])
                ov = jnp.where(lane == l, jnp.sum(acc), ov)
            out_v[pl.ds(k * GC + g * LANES, LANES)] = ov
            return carry

        lax.fori_loop(0, GC // LANES, group, 0)
        if k + 2 < n_chunks:
            handles[slot] = fire(k + 2)

    pltpu.sync_copy(out_v, out_hbm.at[pl.ds(base, b_per_w)])


def kernel(user, item, user_factors, item_factors):
    batch = user.shape[0]
    n_rows, n_factors = user_factors.shape
    nw = NC * NS
    b_per_w = batch // nw
    n_chunks = b_per_w // GC

    mesh = plsc.VectorSubcoreMesh(core_axis_name="c", subcore_axis_name="s")

    out = pl.kernel(
        functools.partial(_body, n_factors, b_per_w, n_chunks),
        out_type=jax.ShapeDtypeStruct((batch,), jnp.float32),
        mesh=mesh,
        scratch_types=[
            pltpu.VMEM((b_per_w,), jnp.int32),
            pltpu.VMEM((b_per_w,), jnp.int32),
            pltpu.VMEM((2, GC, PADF), jnp.float32),
            pltpu.VMEM((2, GC, PADF), jnp.float32),
            pltpu.VMEM((b_per_w,), jnp.float32),
            pltpu.SemaphoreType.DMA,
            pltpu.SemaphoreType.DMA,
        ],
        compiler_params=pltpu.CompilerParams(
            needs_layout_passes=False, use_tc_tiling_on_sc=True),
    )(user.astype(jnp.int32), item.astype(jnp.int32),
      jnp.pad(user_factors, ((0, 0), (0, PADF - n_factors))),
      jnp.pad(item_factors, ((0, 0), (0, PADF - n_factors))))
    return out
